# Initial kernel scaffold; baseline (speedup 1.0000x reference)
import jax
import jax.numpy as jnp
from jax.experimental import pallas as pl


def kernel(cnt_emb, embedding_weight):
    raise NotImplementedError("write your pallas kernel here")


# speedup vs baseline: 1.0423x; 1.0423x over previous
import jax
import jax.numpy as jnp
from jax.experimental import pallas as pl
from jax.experimental.pallas import tpu as pltpu

COMMITMENT_COST = 0.25


def _vq_body(x_ref, emb_t_ref, emb_ref, out_ref, loss_ref, *, grid, n_embed,
             loss_scale):
    i = pl.program_id(0)
    x = x_ref[...]
    emb_t = emb_t_ref[...]
    e_hat = emb_t / jnp.sqrt(jnp.sum(emb_t * emb_t, axis=0, keepdims=True))
    s = jnp.dot(x, e_hat, preferred_element_type=jnp.float32)
    xsq = jnp.sum(x * x, axis=1, keepdims=True)
    esq = jnp.sum(e_hat * e_hat, axis=0, keepdims=True)
    dist = xsq - 2.0 * s + esq
    idx = jnp.argmax(-dist, axis=1)
    s = dist
    ids = jax.lax.broadcasted_iota(jnp.int32, s.shape, 1)
    onehot = (ids == idx[:, None]).astype(jnp.float32)
    q = jnp.dot(onehot, emb_ref[...], preferred_element_type=jnp.float32,
                precision=jax.lax.Precision.HIGHEST)
    d = q - x
    out_ref[...] = x + d
    partial = jnp.sum(d * d)

    @pl.when(i == 0)
    def _init():
        loss_ref[0, 0] = 0.0

    loss_ref[0, 0] += partial

    @pl.when(i == grid - 1)
    def _finish():
        loss_ref[0, 0] = loss_ref[0, 0] * loss_scale


def kernel(cnt_emb, embedding_weight):
    b, t, d = cnt_emb.shape
    n_embed = embedding_weight.shape[0]
    n_rows = b * t
    blk = 1024
    grid = n_rows // blk
    loss_scale = (1.0 + COMMITMENT_COST) / (n_rows * d)

    flat = cnt_emb.reshape(n_rows, d)
    emb_t = embedding_weight.T

    import functools
    body = functools.partial(_vq_body, grid=grid, n_embed=n_embed,
                             loss_scale=loss_scale)
    out, loss = pl.pallas_call(
        body,
        grid=(grid,),
        in_specs=[
            pl.BlockSpec((blk, d), lambda i: (i, 0)),
            pl.BlockSpec((d, n_embed), lambda i: (0, 0)),
            pl.BlockSpec((n_embed, d), lambda i: (0, 0)),
        ],
        out_specs=[
            pl.BlockSpec((blk, d), lambda i: (i, 0)),
            pl.BlockSpec(memory_space=pltpu.SMEM),
        ],
        out_shape=[
            jax.ShapeDtypeStruct((n_rows, d), jnp.float32),
            jax.ShapeDtypeStruct((1, 1), jnp.float32),
        ],
    )(flat, emb_t, embedding_weight)
    return out.reshape(cnt_emb.shape), loss[0, 0]


# speedup vs baseline: 1.4720x; 1.4122x over previous
import jax
import jax.numpy as jnp
from jax.experimental import pallas as pl
from jax.experimental.pallas import tpu as pltpu

COMMITMENT_COST = 0.25


def _vq_body(x_ref, emb_t_ref, emb_ref, out_ref, loss_ref, *, grid, n_embed,
             loss_scale):
    i = pl.program_id(0)
    x = x_ref[...]
    emb_t = emb_t_ref[...]
    e_hat = emb_t / jnp.sqrt(jnp.sum(emb_t * emb_t, axis=0, keepdims=True))
    s = jnp.dot(x, e_hat, preferred_element_type=jnp.float32)
    xsq = jnp.sum(x * x, axis=1, keepdims=True)
    esq = jnp.sum(e_hat * e_hat, axis=0, keepdims=True)
    dist = xsq - 2.0 * s + esq
    idx = jnp.argmax(-dist, axis=1)
    s = dist
    ids = jax.lax.broadcasted_iota(jnp.int32, s.shape, 1)
    onehot = (ids == idx[:, None]).astype(jnp.float32)
    emb = emb_ref[...]
    emb_hi = emb.astype(jnp.bfloat16).astype(jnp.float32)
    emb_lo = emb - emb_hi
    q = (jnp.dot(onehot, emb_hi, preferred_element_type=jnp.float32)
         + jnp.dot(onehot, emb_lo, preferred_element_type=jnp.float32))
    d = q - x
    out_ref[...] = x + d
    partial = jnp.sum(d * d)

    @pl.when(i == 0)
    def _init():
        loss_ref[0, 0] = 0.0

    loss_ref[0, 0] += partial

    @pl.when(i == grid - 1)
    def _finish():
        loss_ref[0, 0] = loss_ref[0, 0] * loss_scale


def kernel(cnt_emb, embedding_weight):
    b, t, d = cnt_emb.shape
    n_embed = embedding_weight.shape[0]
    n_rows = b * t
    blk = 1024
    grid = n_rows // blk
    loss_scale = (1.0 + COMMITMENT_COST) / (n_rows * d)

    flat = cnt_emb.reshape(n_rows, d)
    emb_t = embedding_weight.T

    import functools
    body = functools.partial(_vq_body, grid=grid, n_embed=n_embed,
                             loss_scale=loss_scale)
    out, loss = pl.pallas_call(
        body,
        grid=(grid,),
        in_specs=[
            pl.BlockSpec((blk, d), lambda i: (i, 0)),
            pl.BlockSpec((d, n_embed), lambda i: (0, 0)),
            pl.BlockSpec((n_embed, d), lambda i: (0, 0)),
        ],
        out_specs=[
            pl.BlockSpec((blk, d), lambda i: (i, 0)),
            pl.BlockSpec(memory_space=pltpu.SMEM),
        ],
        out_shape=[
            jax.ShapeDtypeStruct((n_rows, d), jnp.float32),
            jax.ShapeDtypeStruct((1, 1), jnp.float32),
        ],
    )(flat, emb_t, embedding_weight)
    return out.reshape(cnt_emb.shape), loss[0, 0]


# speedup vs baseline: 1.5126x; 1.0275x over previous
import functools

import jax
import jax.numpy as jnp
from jax.experimental import pallas as pl
from jax.experimental.pallas import tpu as pltpu

COMMITMENT_COST = 0.25


def _vq_body(x_ref, emb_t_ref, emb_ref, out_ref, loss_ref, *, nsteps,
             loss_scale):
    i = pl.program_id(0)
    j = pl.program_id(1)
    step = i * pl.num_programs(1) + j
    x = x_ref[0]
    emb_t = emb_t_ref[...]
    e_hat = emb_t / jnp.sqrt(jnp.sum(emb_t * emb_t, axis=0, keepdims=True))
    s = jnp.dot(x, e_hat, preferred_element_type=jnp.float32)
    xsq = jnp.sum(x * x, axis=1, keepdims=True)
    esq = jnp.sum(e_hat * e_hat, axis=0, keepdims=True)
    dist = xsq - 2.0 * s + esq
    idx = jnp.argmax(-dist, axis=1)
    ids = jax.lax.broadcasted_iota(jnp.int32, dist.shape, 1)
    onehot = (ids == idx[:, None]).astype(jnp.float32)
    emb = emb_ref[...]
    emb_hi = emb.astype(jnp.bfloat16).astype(jnp.float32)
    emb_lo = emb - emb_hi
    q = (jnp.dot(onehot, emb_hi, preferred_element_type=jnp.float32)
         + jnp.dot(onehot, emb_lo, preferred_element_type=jnp.float32))
    d = q - x
    out_ref[0] = x + d
    partial = jnp.sum(d * d)

    @pl.when(step == 0)
    def _init():
        loss_ref[0, 0] = 0.0

    loss_ref[0, 0] += partial

    @pl.when(step == nsteps - 1)
    def _finish():
        loss_ref[0, 0] = loss_ref[0, 0] * loss_scale


def kernel(cnt_emb, embedding_weight):
    b, t, d = cnt_emb.shape
    n_embed = embedding_weight.shape[0]
    blk = 1024
    tsteps = t // blk
    nsteps = b * tsteps
    loss_scale = (1.0 + COMMITMENT_COST) / (b * t * d)

    emb_t = embedding_weight.T

    body = functools.partial(_vq_body, nsteps=nsteps, loss_scale=loss_scale)
    out, loss = pl.pallas_call(
        body,
        grid=(b, tsteps),
        in_specs=[
            pl.BlockSpec((1, blk, d), lambda i, j: (i, j, 0)),
            pl.BlockSpec((d, n_embed), lambda i, j: (0, 0)),
            pl.BlockSpec((n_embed, d), lambda i, j: (0, 0)),
        ],
        out_specs=[
            pl.BlockSpec((1, blk, d), lambda i, j: (i, j, 0)),
            pl.BlockSpec(memory_space=pltpu.SMEM),
        ],
        out_shape=[
            jax.ShapeDtypeStruct((b, t, d), jnp.float32),
            jax.ShapeDtypeStruct((1, 1), jnp.float32),
        ],
    )(cnt_emb, emb_t, embedding_weight)
    return out, loss[0, 0]


# speedup vs baseline: 1.7850x; 1.1801x over previous
import functools

import jax
import jax.numpy as jnp
from jax.experimental import pallas as pl
from jax.experimental.pallas import tpu as pltpu

COMMITMENT_COST = 0.25


def _vq_body(x_ref, emb_t_ref, emb_ref, out_ref, loss_ref, *, nsteps,
             loss_scale):
    i = pl.program_id(0)
    j = pl.program_id(1)
    step = i * pl.num_programs(1) + j
    x = x_ref[0]
    emb_t = emb_t_ref[...]
    e_hat = emb_t / jnp.sqrt(jnp.sum(emb_t * emb_t, axis=0, keepdims=True))
    s = jnp.dot(x, e_hat, preferred_element_type=jnp.float32)
    xsq = jnp.sum(x * x, axis=1, keepdims=True)
    esq = jnp.sum(e_hat * e_hat, axis=0, keepdims=True)
    dist = xsq - 2.0 * s + esq
    idx = jnp.argmax(-dist, axis=1)
    ids = jax.lax.broadcasted_iota(jnp.int32, dist.shape, 1)
    onehot = (ids == idx[:, None]).astype(jnp.float32)
    q = jnp.dot(onehot, emb_ref[...], preferred_element_type=jnp.float32)
    d = q - x
    out_ref[0] = x + d
    partial = jnp.sum(d * d)

    @pl.when(step == 0)
    def _init():
        loss_ref[0, 0] = 0.0

    loss_ref[0, 0] += partial

    @pl.when(step == nsteps - 1)
    def _finish():
        loss_ref[0, 0] = loss_ref[0, 0] * loss_scale


def kernel(cnt_emb, embedding_weight):
    b, t, d = cnt_emb.shape
    n_embed = embedding_weight.shape[0]
    blk = 1024
    tsteps = t // blk
    nsteps = b * tsteps
    loss_scale = (1.0 + COMMITMENT_COST) / (b * t * d)

    emb_t = embedding_weight.T

    body = functools.partial(_vq_body, nsteps=nsteps, loss_scale=loss_scale)
    out, loss = pl.pallas_call(
        body,
        grid=(b, tsteps),
        in_specs=[
            pl.BlockSpec((1, blk, d), lambda i, j: (i, j, 0)),
            pl.BlockSpec((d, n_embed), lambda i, j: (0, 0)),
            pl.BlockSpec((n_embed, d), lambda i, j: (0, 0)),
        ],
        out_specs=[
            pl.BlockSpec((1, blk, d), lambda i, j: (i, j, 0)),
            pl.BlockSpec(memory_space=pltpu.SMEM),
        ],
        out_shape=[
            jax.ShapeDtypeStruct((b, t, d), jnp.float32),
            jax.ShapeDtypeStruct((1, 1), jnp.float32),
        ],
    )(cnt_emb, emb_t, embedding_weight)
    return out, loss[0, 0]


# speedup vs baseline: 1.9241x; 1.0780x over previous
import functools

import jax
import jax.numpy as jnp
from jax.experimental import pallas as pl
from jax.experimental.pallas import tpu as pltpu

COMMITMENT_COST = 0.25


def _vq_body(x_ref, emb_t_ref, emb_ref, out_ref, loss_ref, *, nsteps,
             loss_scale):
    i = pl.program_id(0)
    j = pl.program_id(1)
    step = i * pl.num_programs(1) + j
    x = x_ref[0]
    emb_t = emb_t_ref[...]
    e_hat = emb_t / jnp.sqrt(jnp.sum(emb_t * emb_t, axis=0, keepdims=True))
    s = jnp.dot(x, e_hat, preferred_element_type=jnp.float32)
    xsq = jnp.sum(x * x, axis=1, keepdims=True)
    esq = jnp.sum(e_hat * e_hat, axis=0, keepdims=True)
    dist = xsq - 2.0 * s + esq
    idx = jnp.argmax(-dist, axis=1)
    ids = jax.lax.broadcasted_iota(jnp.int32, dist.shape, 1)
    onehot = (ids == idx[:, None]).astype(jnp.float32)
    q = jnp.dot(onehot, emb_ref[...], preferred_element_type=jnp.float32)
    d = q - x
    out_ref[0] = x + d
    partial = jnp.sum(d * d)

    @pl.when(step == 0)
    def _init():
        loss_ref[0, 0] = 0.0

    loss_ref[0, 0] += partial

    @pl.when(step == nsteps - 1)
    def _finish():
        loss_ref[0, 0] = loss_ref[0, 0] * loss_scale


def kernel(cnt_emb, embedding_weight):
    b, t, d = cnt_emb.shape
    n_embed = embedding_weight.shape[0]
    blk = 2048
    tsteps = t // blk
    nsteps = b * tsteps
    loss_scale = (1.0 + COMMITMENT_COST) / (b * t * d)

    emb_t = embedding_weight.T

    body = functools.partial(_vq_body, nsteps=nsteps, loss_scale=loss_scale)
    out, loss = pl.pallas_call(
        body,
        grid=(b, tsteps),
        in_specs=[
            pl.BlockSpec((1, blk, d), lambda i, j: (i, j, 0)),
            pl.BlockSpec((d, n_embed), lambda i, j: (0, 0)),
            pl.BlockSpec((n_embed, d), lambda i, j: (0, 0)),
        ],
        out_specs=[
            pl.BlockSpec((1, blk, d), lambda i, j: (i, j, 0)),
            pl.BlockSpec(memory_space=pltpu.SMEM),
        ],
        out_shape=[
            jax.ShapeDtypeStruct((b, t, d), jnp.float32),
            jax.ShapeDtypeStruct((1, 1), jnp.float32),
        ],
    )(cnt_emb, emb_t, embedding_weight)
    return out, loss[0, 0]


# speedup vs baseline: 2.0196x; 1.0496x over previous
import functools

import jax
import jax.numpy as jnp
from jax.experimental import pallas as pl
from jax.experimental.pallas import tpu as pltpu

COMMITMENT_COST = 0.25


def _vq_body(x_ref, emb_t_ref, emb_ref, out_ref, loss_ref, *, nsteps,
             loss_scale):
    i = pl.program_id(0)
    j = pl.program_id(1)
    step = i * pl.num_programs(1) + j
    x = x_ref[0]
    emb_t = emb_t_ref[...]
    e_hat = emb_t / jnp.sqrt(jnp.sum(emb_t * emb_t, axis=0, keepdims=True))
    s = jnp.dot(x, e_hat, preferred_element_type=jnp.float32)
    xsq = jnp.sum(x * x, axis=1, keepdims=True)
    esq = jnp.sum(e_hat * e_hat, axis=0, keepdims=True)
    dist = xsq - 2.0 * s + esq
    idx = jnp.argmax(-dist, axis=1)
    ids = jax.lax.broadcasted_iota(jnp.int32, dist.shape, 1)
    onehot = (ids == idx[:, None]).astype(jnp.float32)
    q = jnp.dot(onehot, emb_ref[...], preferred_element_type=jnp.float32)
    d = q - x
    out_ref[0] = x + d
    partial = jnp.sum(d * d)

    @pl.when(step == 0)
    def _init():
        loss_ref[0, 0] = 0.0

    loss_ref[0, 0] += partial

    @pl.when(step == nsteps - 1)
    def _finish():
        loss_ref[0, 0] = loss_ref[0, 0] * loss_scale


def kernel(cnt_emb, embedding_weight):
    b, t, d = cnt_emb.shape
    n_embed = embedding_weight.shape[0]
    blk = 4096
    tsteps = t // blk
    nsteps = b * tsteps
    loss_scale = (1.0 + COMMITMENT_COST) / (b * t * d)

    emb_t = embedding_weight.T

    body = functools.partial(_vq_body, nsteps=nsteps, loss_scale=loss_scale)
    out, loss = pl.pallas_call(
        body,
        grid=(b, tsteps),
        in_specs=[
            pl.BlockSpec((1, blk, d), lambda i, j: (i, j, 0)),
            pl.BlockSpec((d, n_embed), lambda i, j: (0, 0)),
            pl.BlockSpec((n_embed, d), lambda i, j: (0, 0)),
        ],
        out_specs=[
            pl.BlockSpec((1, blk, d), lambda i, j: (i, j, 0)),
            pl.BlockSpec(memory_space=pltpu.SMEM),
        ],
        out_shape=[
            jax.ShapeDtypeStruct((b, t, d), jnp.float32),
            jax.ShapeDtypeStruct((1, 1), jnp.float32),
        ],
    )(cnt_emb, emb_t, embedding_weight)
    return out, loss[0, 0]


# speedup vs baseline: 2.0347x; 1.0075x over previous
import functools

import jax
import jax.numpy as jnp
from jax.experimental import pallas as pl
from jax.experimental.pallas import tpu as pltpu

COMMITMENT_COST = 0.25


def _vq_body(x_ref, emb_t_ref, emb_ref, out_ref, loss_ref, *, nsteps,
             loss_scale):
    i = pl.program_id(0)
    j = pl.program_id(1)
    step = i * pl.num_programs(1) + j
    x = x_ref[0]
    emb_t = emb_t_ref[...]
    e_hat = emb_t / jnp.sqrt(jnp.sum(emb_t * emb_t, axis=0, keepdims=True))
    s = jnp.dot(x, e_hat, preferred_element_type=jnp.float32)
    xsq = jnp.sum(x * x, axis=1, keepdims=True)
    esq = jnp.sum(e_hat * e_hat, axis=0, keepdims=True)
    nd = (2.0 * s - xsq) - esq
    idx = jnp.argmax(nd, axis=1)
    ids = jax.lax.broadcasted_iota(jnp.int32, nd.shape, 1)
    onehot = (ids == idx[:, None]).astype(jnp.bfloat16)
    q = jnp.dot(onehot, emb_ref[...].astype(jnp.bfloat16),
                preferred_element_type=jnp.float32)
    d = q - x
    out_ref[0] = x + d
    partial = jnp.sum(d * d)

    @pl.when(step == 0)
    def _init():
        loss_ref[0, 0] = 0.0

    loss_ref[0, 0] += partial

    @pl.when(step == nsteps - 1)
    def _finish():
        loss_ref[0, 0] = loss_ref[0, 0] * loss_scale


def kernel(cnt_emb, embedding_weight):
    b, t, d = cnt_emb.shape
    n_embed = embedding_weight.shape[0]
    blk = 4096
    tsteps = t // blk
    nsteps = b * tsteps
    loss_scale = (1.0 + COMMITMENT_COST) / (b * t * d)

    emb_t = embedding_weight.T

    body = functools.partial(_vq_body, nsteps=nsteps, loss_scale=loss_scale)
    out, loss = pl.pallas_call(
        body,
        grid=(b, tsteps),
        in_specs=[
            pl.BlockSpec((1, blk, d), lambda i, j: (i, j, 0)),
            pl.BlockSpec((d, n_embed), lambda i, j: (0, 0)),
            pl.BlockSpec((n_embed, d), lambda i, j: (0, 0)),
        ],
        out_specs=[
            pl.BlockSpec((1, blk, d), lambda i, j: (i, j, 0)),
            pl.BlockSpec(memory_space=pltpu.SMEM),
        ],
        out_shape=[
            jax.ShapeDtypeStruct((b, t, d), jnp.float32),
            jax.ShapeDtypeStruct((1, 1), jnp.float32),
        ],
    )(cnt_emb, emb_t, embedding_weight)
    return out, loss[0, 0]


# speedup vs baseline: 2.1510x; 1.0571x over previous
import functools

import jax
import jax.numpy as jnp
from jax.experimental import pallas as pl
from jax.experimental.pallas import tpu as pltpu

COMMITMENT_COST = 0.25


def _vq_body(x_ref, emb_t_ref, emb_ref, out_ref, loss_ref, *, nsteps,
             loss_scale):
    i = pl.program_id(0)
    j = pl.program_id(1)
    step = i * pl.num_programs(1) + j
    x = x_ref[0]
    emb_t = emb_t_ref[...]
    e_hat = emb_t / jnp.sqrt(jnp.sum(emb_t * emb_t, axis=0, keepdims=True))
    s = jnp.dot(x, e_hat, preferred_element_type=jnp.float32)
    xsq = jnp.sum(x * x, axis=1, keepdims=True)
    esq = jnp.sum(e_hat * e_hat, axis=0, keepdims=True)
    nd = (2.0 * s - xsq) - esq
    m = jnp.max(nd, axis=1, keepdims=True)
    ids = jax.lax.broadcasted_iota(jnp.int32, nd.shape, 1)
    idx = jnp.min(jnp.where(nd == m, ids, jnp.int32(nd.shape[1])), axis=1,
                  keepdims=True)
    onehot = (ids == idx).astype(jnp.bfloat16)
    q = jnp.dot(onehot, emb_ref[...].astype(jnp.bfloat16),
                preferred_element_type=jnp.float32)
    d = q - x
    out_ref[0] = x + d
    partial = jnp.sum(d * d)

    @pl.when(step == 0)
    def _init():
        loss_ref[0, 0] = 0.0

    loss_ref[0, 0] += partial

    @pl.when(step == nsteps - 1)
    def _finish():
        loss_ref[0, 0] = loss_ref[0, 0] * loss_scale


def kernel(cnt_emb, embedding_weight):
    b, t, d = cnt_emb.shape
    n_embed = embedding_weight.shape[0]
    blk = 4096
    tsteps = t // blk
    nsteps = b * tsteps
    loss_scale = (1.0 + COMMITMENT_COST) / (b * t * d)

    emb_t = embedding_weight.T

    body = functools.partial(_vq_body, nsteps=nsteps, loss_scale=loss_scale)
    out, loss = pl.pallas_call(
        body,
        grid=(b, tsteps),
        in_specs=[
            pl.BlockSpec((1, blk, d), lambda i, j: (i, j, 0)),
            pl.BlockSpec((d, n_embed), lambda i, j: (0, 0)),
            pl.BlockSpec((n_embed, d), lambda i, j: (0, 0)),
        ],
        out_specs=[
            pl.BlockSpec((1, blk, d), lambda i, j: (i, j, 0)),
            pl.BlockSpec(memory_space=pltpu.SMEM),
        ],
        out_shape=[
            jax.ShapeDtypeStruct((b, t, d), jnp.float32),
            jax.ShapeDtypeStruct((1, 1), jnp.float32),
        ],
    )(cnt_emb, emb_t, embedding_weight)
    return out, loss[0, 0]


# speedup vs baseline: 2.2989x; 1.0688x over previous
import functools

import jax
import jax.numpy as jnp
from jax.experimental import pallas as pl
from jax.experimental.pallas import tpu as pltpu

COMMITMENT_COST = 0.25


def _vq_body(x_ref, emb_t_ref, emb_aug_ref, out_ref, loss_ref, *, nsteps,
             loss_scale):
    i = pl.program_id(0)
    j = pl.program_id(1)
    step = i * pl.num_programs(1) + j
    x = x_ref[0]
    emb_t = emb_t_ref[...]
    e_hat = emb_t / jnp.sqrt(jnp.sum(emb_t * emb_t, axis=0, keepdims=True))
    s = jnp.dot(x, e_hat, preferred_element_type=jnp.float32)
    xsq = jnp.sum(x * x, axis=1, keepdims=True)
    esq = jnp.sum(e_hat * e_hat, axis=0, keepdims=True)
    nd = (2.0 * s - xsq) - esq
    m = jnp.max(nd, axis=1, keepdims=True)
    onehot = (nd == m).astype(jnp.bfloat16)
    qc = jnp.dot(onehot, emb_aug_ref[...], preferred_element_type=jnp.float32)
    nmel = x.shape[1]
    q = qc[:, :nmel] / qc[:, nmel:nmel + 1]
    d = q - x
    out_ref[0] = x + d
    partial = jnp.sum(d * d)

    @pl.when(step == 0)
    def _init():
        loss_ref[0, 0] = 0.0

    loss_ref[0, 0] += partial

    @pl.when(step == nsteps - 1)
    def _finish():
        loss_ref[0, 0] = loss_ref[0, 0] * loss_scale


def kernel(cnt_emb, embedding_weight):
    b, t, d = cnt_emb.shape
    n_embed = embedding_weight.shape[0]
    blk = 4096
    tsteps = t // blk
    nsteps = b * tsteps
    loss_scale = (1.0 + COMMITMENT_COST) / (b * t * d)

    emb_t = embedding_weight.T
    emb_aug = jnp.concatenate(
        [embedding_weight, jnp.ones((n_embed, 1), jnp.float32)], axis=1
    ).astype(jnp.bfloat16)

    body = functools.partial(_vq_body, nsteps=nsteps, loss_scale=loss_scale)
    out, loss = pl.pallas_call(
        body,
        grid=(b, tsteps),
        in_specs=[
            pl.BlockSpec((1, blk, d), lambda i, j: (i, j, 0)),
            pl.BlockSpec((d, n_embed), lambda i, j: (0, 0)),
            pl.BlockSpec((n_embed, d + 1), lambda i, j: (0, 0)),
        ],
        out_specs=[
            pl.BlockSpec((1, blk, d), lambda i, j: (i, j, 0)),
            pl.BlockSpec(memory_space=pltpu.SMEM),
        ],
        out_shape=[
            jax.ShapeDtypeStruct((b, t, d), jnp.float32),
            jax.ShapeDtypeStruct((1, 1), jnp.float32),
        ],
    )(cnt_emb, emb_t, emb_aug)
    return out, loss[0, 0]
